# baseline (speedup 1.0000x reference)
import functools

import jax
import jax.numpy as jnp
from jax import lax
from jax.experimental import pallas as pl
from jax.experimental.pallas import tpu as pltpu
from jax.experimental.pallas import tpu_sc as plsc

B = 16384
D = 32
L = 16
NC = 2
NS = 16
NW = NC * NS
BPW = B // NW
G = BPW // L


def _sc_body(k_hbm, table_hbm, wb_hbm, out_hbm, idx_v, rows_v, wb_v, out_v, sem):
    wid = lax.axis_index("s") * NC + lax.axis_index("c")
    base = wid * BPW

    pltpu.sync_copy(k_hbm.at[pl.ds(base, BPW)], idx_v)
    pltpu.sync_copy(wb_hbm, wb_v)
    pltpu.async_copy(table_hbm.at[idx_v], rows_v, sem).wait()

    w0 = wb_v[0]
    w1 = wb_v[1]
    bias = wb_v[2]
    lane = lax.iota(jnp.int32, L)
    rots = [(lane + sh) % L for sh in (8, 4, 2, 1)]

    def group_body(g, carry):
        acc = bias
        for r0 in range(L):
            r = g * L + r0
            x0 = rows_v[r, pl.ds(0, L)]
            x1 = rows_v[r, pl.ds(L, L)]
            t = (x0 * w0) / (1.0 + jnp.exp(-x0)) + (x1 * w1) / (
                1.0 + jnp.exp(-x1)
            )
            for rot in rots:
                t = t + t.at[rot].get(mode="promise_in_bounds")
            acc = jnp.where(lane == r0, acc + t, acc)
        out_v[pl.ds(g * L, L)] = 1.0 / (1.0 + jnp.exp(-acc))
        return carry

    lax.fori_loop(0, G, group_body, 0)
    pltpu.sync_copy(out_v, out_hbm.at[pl.ds(base, BPW)])


_sc_kernel = functools.partial(
    pl.kernel,
    out_type=jax.ShapeDtypeStruct((B,), jnp.float32),
    mesh=plsc.VectorSubcoreMesh(core_axis_name="c", subcore_axis_name="s"),
    compiler_params=pltpu.CompilerParams(use_tc_tiling_on_sc=False),
    scratch_types=[
        pltpu.VMEM((BPW,), jnp.int32),
        pltpu.VMEM((BPW, D), jnp.float32),
        pltpu.VMEM((3, L), jnp.float32),
        pltpu.VMEM((BPW,), jnp.float32),
        pltpu.SemaphoreType.DMA,
    ],
)(_sc_body)


def kernel(k, emb_table, W, b):
    wf = W.reshape(D)
    wb = jnp.stack(
        [wf[:L], wf[L:], jnp.broadcast_to(b.reshape(1), (L,))]
    )
    out = _sc_kernel(k.astype(jnp.int32), emb_table, wb)
    return out.reshape(B, 1)


# speedup vs baseline: 1.3343x; 1.3343x over previous
import functools

import jax
import jax.numpy as jnp
from jax import lax
from jax.experimental import pallas as pl
from jax.experimental.pallas import tpu as pltpu
from jax.experimental.pallas import tpu_sc as plsc

B = 16384
D = 32
L = 16
NC = 2
NS = 16
FPC = D // NC
KPT = B // NS
J = KPT // L


def _sc_body(k_hbm, tt_hbm, w_hbm, out_hbm, idx_v, wv, col_all, out_v, sem):
    c = lax.axis_index("c")
    s = lax.axis_index("s")
    base = s * KPT

    pltpu.sync_copy(k_hbm.at[pl.ds(base, KPT)], idx_v)
    pltpu.sync_copy(w_hbm.at[pl.ds(c * FPC, FPC)], wv)

    copies = []
    for f in range(FPC):
        copies.append(
            pltpu.async_copy(tt_hbm.at[c * FPC + f].at[idx_v], col_all.at[f], sem)
        )
    for cp in copies:
        cp.wait()

    def chunk_body(j, carry):
        acc = jnp.zeros((L,), jnp.float32)
        for f in range(FPC):
            x = col_all[f, pl.ds(j * L, L)]
            w = wv[f]
            acc = acc + (x * w) / (1.0 + jnp.exp(-x))
        out_v[pl.ds(j * L, L)] = acc
        return carry

    lax.fori_loop(0, J, chunk_body, 0)
    pltpu.sync_copy(out_v, out_hbm.at[c, pl.ds(base, KPT)])


_sc_kernel = functools.partial(
    pl.kernel,
    out_type=jax.ShapeDtypeStruct((NC, B), jnp.float32),
    mesh=plsc.VectorSubcoreMesh(core_axis_name="c", subcore_axis_name="s"),
    compiler_params=pltpu.CompilerParams(use_tc_tiling_on_sc=False),
    scratch_types=[
        pltpu.VMEM((KPT,), jnp.int32),
        pltpu.VMEM((FPC, L), jnp.float32),
        pltpu.VMEM((FPC, KPT), jnp.float32),
        pltpu.VMEM((KPT,), jnp.float32),
        pltpu.SemaphoreType.DMA,
    ],
)(_sc_body)


def kernel(k, emb_table, W, b):
    tt = emb_table.T
    wbb = jnp.broadcast_to(W.reshape(D, 1), (D, L))
    parts = _sc_kernel(k.astype(jnp.int32), tt, wbb)
    out = jax.nn.sigmoid(parts[0] + parts[1] + b[0])
    return out.reshape(B, 1)


# speedup vs baseline: 1.4285x; 1.0706x over previous
import functools

import jax
import jax.numpy as jnp
from jax import lax
from jax.experimental import pallas as pl
from jax.experimental.pallas import tpu as pltpu
from jax.experimental.pallas import tpu_sc as plsc

B = 16384
D = 32
V = 100000
L = 16
NC = 2
NS = 16
FPC = D // NC
KPT = B // NS
J = KPT // L


def _sc_body(k_hbm, tt_hbm, w_hbm, out_hbm, spm, idx_v, wv, col_all, out_v,
             sem_stage, sem_g):
    c = lax.axis_index("c")
    s = lax.axis_index("s")
    base = s * KPT

    stage = pltpu.async_copy(tt_hbm.at[c * FPC + s], spm.at[s], sem_stage)
    pltpu.sync_copy(k_hbm.at[pl.ds(base, KPT)], idx_v)
    pltpu.sync_copy(w_hbm.at[pl.ds(c * FPC, FPC)], wv)
    stage.wait()
    plsc.subcore_barrier()

    copies = [
        pltpu.async_copy(spm.at[f].at[idx_v], col_all.at[f], sem_g)
        for f in range(FPC)
    ]

    for f in range(FPC):
        copies[f].wait()

        def body(j, carry, f=f):
            x = col_all[f, pl.ds(j * L, L)]
            w = wv[f]
            t = (x * w) / (1.0 + jnp.exp(-x))
            if f == 0:
                out_v[pl.ds(j * L, L)] = t
            else:
                out_v[pl.ds(j * L, L)] += t
            return carry

        lax.fori_loop(0, J, body, 0)

    pltpu.sync_copy(out_v, out_hbm.at[c, pl.ds(base, KPT)])


_sc_kernel = functools.partial(
    pl.kernel,
    out_type=jax.ShapeDtypeStruct((NC, B), jnp.float32),
    mesh=plsc.VectorSubcoreMesh(core_axis_name="c", subcore_axis_name="s"),
    compiler_params=pltpu.CompilerParams(use_tc_tiling_on_sc=False),
    scratch_types=[
        pltpu.VMEM_SHARED((FPC, V), jnp.float32),
        pltpu.VMEM((KPT,), jnp.int32),
        pltpu.VMEM((FPC, L), jnp.float32),
        pltpu.VMEM((FPC, KPT), jnp.float32),
        pltpu.VMEM((KPT,), jnp.float32),
        pltpu.SemaphoreType.DMA,
        pltpu.SemaphoreType.DMA,
    ],
)(_sc_body)


def kernel(k, emb_table, W, b):
    tt = emb_table.T
    wbb = jnp.broadcast_to(W.reshape(D, 1), (D, L))
    parts = _sc_kernel(k.astype(jnp.int32), tt, wbb)
    out = jax.nn.sigmoid(parts[0] + parts[1] + b[0])
    return out.reshape(B, 1)
